# baseline (speedup 1.0000x reference)
import jax
import jax.numpy as jnp
from jax import lax
from jax.experimental import pallas as pl
from jax.experimental.pallas import tpu as pltpu

NC = 19
H = 512
W = 512
R = 128
LANES = 128


def _seg_loss_kernel(x_ref, out_ref, s_acc, n_acc, carry_has, carry_v):
    b = pl.program_id(0)
    k = pl.program_id(1)
    nk = pl.num_programs(1)

    @pl.when(k == 0)
    def _init():
        s_acc[...] = jnp.zeros_like(s_acc)
        n_acc[...] = jnp.zeros_like(n_acc)
        carry_has[...] = jnp.zeros_like(carry_has)
        carry_v[...] = jnp.zeros_like(carry_v)

    best = x_ref[0, 0]
    seg = jnp.zeros((R, W), jnp.int32)
    for c in range(1, NC):
        xc = x_ref[0, c]
        m = xc > best
        best = jnp.where(m, xc, best)
        seg = jnp.where(m, c, seg)

    col = lax.broadcasted_iota(jnp.int32, (R, W), 1)
    cnts, fsts, lsts = [], [], []
    for c in range(1, NC):
        mask = seg == c
        cnts.append(jnp.sum(mask.astype(jnp.int32), axis=1, keepdims=True))
        fsts.append(jnp.min(jnp.where(mask, col, W), axis=1, keepdims=True))
        lsts.append(jnp.max(jnp.where(mask, col, -1), axis=1, keepdims=True))
    pad = [jnp.zeros((R, LANES - (NC - 1)), jnp.int32)]
    cnt = jnp.concatenate(cnts + pad, axis=1)
    fst = jnp.concatenate(fsts + pad, axis=1).astype(jnp.float32)
    lst = jnp.concatenate(lsts + pad, axis=1).astype(jnp.float32)

    e = (cnt > 0).astype(jnp.float32)
    rowg = (k * R + lax.broadcasted_iota(jnp.int32, (R, LANES), 0)).astype(
        jnp.float32)
    within = jnp.sum(e * (lst - fst), axis=0, keepdims=True)
    nch = jnp.sum(cnt, axis=0, keepdims=True).astype(jnp.float32)

    v_first = fst - rowg
    v_last = lst - rowg

    ff = e * v_last
    fv = e
    sh = 1
    while sh < R:
        zf = jnp.zeros((sh, LANES), jnp.float32)
        ffs = jnp.concatenate([zf, ff[: R - sh]], axis=0)
        fvs = jnp.concatenate([zf, fv[: R - sh]], axis=0)
        ff = jnp.where(fv > 0.0, ff, ffs)
        fv = jnp.maximum(fv, fvs)
        sh *= 2

    z1 = jnp.zeros((1, LANES), jnp.float32)
    pf = jnp.concatenate([z1, ff[: R - 1]], axis=0)
    pv = jnp.concatenate([z1, fv[: R - 1]], axis=0)
    cv = jnp.broadcast_to(carry_v[...], (R, LANES))
    ch = jnp.broadcast_to(carry_has[...], (R, LANES))
    prev_v = jnp.where(pv > 0.0, pf, cv)
    prev_ok = jnp.maximum(pv, ch)
    cross = jnp.sum(e * prev_ok * jnp.abs(v_first - prev_v), axis=0,
                    keepdims=True)

    s_acc[...] += within + cross
    n_acc[...] += nch
    last_ok = fv[R - 1 : R]
    carry_v[...] = jnp.where(last_ok > 0.0, ff[R - 1 : R], carry_v[...])
    carry_has[...] = jnp.maximum(carry_has[...], last_ok)

    @pl.when(k == nk - 1)
    def _finish():
        nf = n_acc[...]
        s = s_acc[...]
        mean = s / jnp.maximum(nf - 1.0, 1.0)
        contrib = jnp.where(nf >= 2.0, mean / (nf + 1.0), 0.0)
        out_ref[0, 0, 0] = jnp.sum(contrib)


def kernel(logits, labels):
    del labels
    bs = logits.shape[0]
    out = pl.pallas_call(
        _seg_loss_kernel,
        grid=(bs, H // R),
        in_specs=[
            pl.BlockSpec((1, NC, R, W), lambda b, k: (b, 0, k, 0)),
        ],
        out_specs=pl.BlockSpec((1, 1, 1), lambda b, k: (b, 0, 0),
                               memory_space=pltpu.SMEM),
        out_shape=jax.ShapeDtypeStruct((bs, 1, 1), jnp.float32),
        scratch_shapes=[
            pltpu.VMEM((1, LANES), jnp.float32),
            pltpu.VMEM((1, LANES), jnp.float32),
            pltpu.VMEM((1, LANES), jnp.float32),
            pltpu.VMEM((1, LANES), jnp.float32),
        ],
        compiler_params=pltpu.CompilerParams(
            dimension_semantics=("parallel", "arbitrary")),
    )(logits)
    return jnp.sum(out)


# speedup vs baseline: 1.5109x; 1.5109x over previous
import jax
import jax.numpy as jnp
from jax import lax
from jax.experimental import pallas as pl
from jax.experimental.pallas import tpu as pltpu

NC = 19
H = 512
W = 512
R = 128
LANES = 128


def _seg_loss_kernel(x_ref, out_ref, s_acc, n_acc, carry_has, carry_v):
    b = pl.program_id(0)
    k = pl.program_id(1)
    nk = pl.num_programs(1)

    @pl.when(k == 0)
    def _init():
        s_acc[...] = jnp.zeros_like(s_acc)
        n_acc[...] = jnp.zeros_like(n_acc)
        carry_has[...] = jnp.zeros_like(carry_has)
        carry_v[...] = jnp.zeros_like(carry_v)

    best = x_ref[0, 0]
    seg = jnp.zeros((R, W), jnp.int32)
    for c in range(1, NC):
        xc = x_ref[0, c]
        m = xc > best
        best = jnp.where(m, xc, best)
        seg = jnp.where(m, c, seg)

    colf = lax.broadcasted_iota(jnp.int32, (R, W), 1).astype(jnp.float32)
    one = jnp.ones((R, W), jnp.float32)
    zero = jnp.zeros((R, W), jnp.float32)
    cnts, fsts, lsts = [], [], []
    for c in range(1, NC):
        mask = seg == c
        cnts.append(jnp.sum(jnp.where(mask, one, zero), axis=1,
                            keepdims=True))
        fsts.append(jnp.min(jnp.where(mask, colf, jnp.float32(W)), axis=1,
                            keepdims=True))
        lsts.append(jnp.max(jnp.where(mask, colf, jnp.float32(-1.0)), axis=1,
                            keepdims=True))
    pad = [jnp.zeros((R, LANES - (NC - 1)), jnp.float32)]
    cnt = jnp.concatenate(cnts + pad, axis=1)
    fst = jnp.concatenate(fsts + pad, axis=1)
    lst = jnp.concatenate(lsts + pad, axis=1)

    e = (cnt > 0.0).astype(jnp.float32)
    rowg = (k * R + lax.broadcasted_iota(jnp.int32, (R, LANES), 0)).astype(
        jnp.float32)
    within = jnp.sum(e * (lst - fst), axis=0, keepdims=True)
    nch = jnp.sum(cnt, axis=0, keepdims=True).astype(jnp.float32)

    v_first = fst - rowg
    v_last = lst - rowg

    ff = e * v_last
    fv = e
    sh = 1
    while sh < R:
        zf = jnp.zeros((sh, LANES), jnp.float32)
        ffs = jnp.concatenate([zf, ff[: R - sh]], axis=0)
        fvs = jnp.concatenate([zf, fv[: R - sh]], axis=0)
        ff = jnp.where(fv > 0.0, ff, ffs)
        fv = jnp.maximum(fv, fvs)
        sh *= 2

    z1 = jnp.zeros((1, LANES), jnp.float32)
    pf = jnp.concatenate([z1, ff[: R - 1]], axis=0)
    pv = jnp.concatenate([z1, fv[: R - 1]], axis=0)
    cv = jnp.broadcast_to(carry_v[...], (R, LANES))
    ch = jnp.broadcast_to(carry_has[...], (R, LANES))
    prev_v = jnp.where(pv > 0.0, pf, cv)
    prev_ok = jnp.maximum(pv, ch)
    cross = jnp.sum(e * prev_ok * jnp.abs(v_first - prev_v), axis=0,
                    keepdims=True)

    s_acc[...] += within + cross
    n_acc[...] += nch
    last_ok = fv[R - 1 : R]
    carry_v[...] = jnp.where(last_ok > 0.0, ff[R - 1 : R], carry_v[...])
    carry_has[...] = jnp.maximum(carry_has[...], last_ok)

    @pl.when(k == nk - 1)
    def _finish():
        nf = n_acc[...]
        s = s_acc[...]
        mean = s / jnp.maximum(nf - 1.0, 1.0)
        contrib = jnp.where(nf >= 2.0, mean / (nf + 1.0), 0.0)
        bt = jnp.sum(contrib)

        @pl.when(b == 0)
        def _set():
            out_ref[0, 0] = bt

        @pl.when(b != 0)
        def _add():
            out_ref[0, 0] += bt


def kernel(logits, labels):
    del labels
    bs = logits.shape[0]
    out = pl.pallas_call(
        _seg_loss_kernel,
        grid=(bs, H // R),
        in_specs=[
            pl.BlockSpec((1, NC, R, W), lambda b, k: (b, 0, k, 0)),
        ],
        out_specs=pl.BlockSpec(memory_space=pltpu.SMEM),
        out_shape=jax.ShapeDtypeStruct((1, 1), jnp.float32),
        scratch_shapes=[
            pltpu.VMEM((1, LANES), jnp.float32),
            pltpu.VMEM((1, LANES), jnp.float32),
            pltpu.VMEM((1, LANES), jnp.float32),
            pltpu.VMEM((1, LANES), jnp.float32),
        ],
    )(logits)
    return out[0, 0]
